# baseline (speedup 1.0000x reference)
import math
from functools import partial

import jax
import jax.numpy as jnp
from jax import lax
from jax.experimental import pallas as pl
from jax.experimental.pallas import tpu as pltpu

BN_EPS = 1e-5
VMEM_LIMIT = 48 * 1024 * 1024


def _im2col_dot(xpad_ref, xcol_ref, w_ref, H, W):
    W2 = W + 2
    M = H * W2
    Cin = xpad_ref.shape[-1]
    for t in range(9):
        off = (t // 3) * W2 + (t % 3)
        xcol_ref[:, t * Cin:(t + 1) * Cin] = xpad_ref[pl.ds(off, M), :]
    return jnp.dot(xcol_ref[...], w_ref[...],
                   preferred_element_type=jnp.float32)


def _store_out_and_stats(acc, H, W, y_ref, st_ref):
    W2 = W + 2
    M = H * W2
    r = lax.broadcasted_iota(jnp.int32, (M, 1), 0)
    ym = jnp.where((r % W2) < W, acc, 0.0)
    y_ref[...] = ym.astype(y_ref.dtype)
    st_ref[0:1, :] = jnp.sum(ym, axis=0, keepdims=True)
    st_ref[1:2, :] = jnp.sum(ym * ym, axis=0, keepdims=True)


def _ss_kernel(st_ref, g_ref, b_ref, out_ref, *, cnt):
    s1 = jnp.sum(st_ref[:, 0, :], axis=0, keepdims=True)
    s2 = jnp.sum(st_ref[:, 1, :], axis=0, keepdims=True)
    mean = s1 / cnt
    var = jnp.maximum(s2 / cnt - mean * mean, 0.0)
    scale = g_ref[...] * lax.rsqrt(var + BN_EPS)
    out_ref[0:1, :] = scale
    out_ref[1:2, :] = b_ref[...] - mean * scale


def _conv1_kernel(x_ref, w_ref, y_ref, st_ref, xcol_ref, *, H, W):
    acc = _im2col_dot(x_ref, xcol_ref, w_ref, H, W)
    _store_out_and_stats(acc, H, W, y_ref, st_ref)


def _pool_conv_kernel(prev_ref, ss_ref, w_ref, y_ref, st_ref,
                      act_ref, rmax_ref, xpad_ref, xcol_ref,
                      *, Hp, Wp, Hc, Wc):
    scale = ss_ref[0:1, :]
    shift = ss_ref[1:2, :]
    act_ref[...] = jnp.maximum(
        prev_ref[...].astype(jnp.float32) * scale + shift, 0.0)

    cm = act_ref[:, pl.ds(0, Wc, stride=2), :]
    cm = jnp.maximum(cm, act_ref[:, pl.ds(1, Wc, stride=2), :])
    cm = jnp.maximum(cm, act_ref[:, pl.ds(2, Wc, stride=2), :])
    rmax_ref[...] = cm
    pooled = rmax_ref[pl.ds(0, Hc, stride=2), :, :]
    pooled = jnp.maximum(pooled, rmax_ref[pl.ds(1, Hc, stride=2), :, :])
    pooled = jnp.maximum(pooled, rmax_ref[pl.ds(2, Hc, stride=2), :, :])
    pooled = pooled.astype(jnp.bfloat16)

    W2c = Wc + 2

    @pl.when(pl.program_id(0) == 0)
    def _zero_pad():
        xpad_ref[...] = jnp.zeros_like(xpad_ref)

    for h in range(Hc):
        xpad_ref[pl.ds((h + 1) * W2c + 1, Wc), :] = pooled[h]

    acc = _im2col_dot(xpad_ref, xcol_ref, w_ref, Hc, Wc)
    _store_out_and_stats(acc, Hc, Wc, y_ref, st_ref)


def _conv_kernel(prev_ref, ss_ref, w_ref, y_ref, st_ref, xpad_ref, xcol_ref,
                 *, H, W):
    W2 = W + 2
    M = H * W2
    scale = ss_ref[0:1, :]
    shift = ss_ref[1:2, :]
    r = lax.broadcasted_iota(jnp.int32, (M, 1), 0)
    norm = jnp.where(
        (r % W2) < W,
        jnp.maximum(prev_ref[...].astype(jnp.float32) * scale + shift, 0.0),
        0.0).astype(jnp.bfloat16)

    @pl.when(pl.program_id(0) == 0)
    def _zero_pad():
        xpad_ref[...] = jnp.zeros_like(xpad_ref)

    xpad_ref[pl.ds(W2 + 1, M), :] = norm
    acc = _im2col_dot(xpad_ref, xcol_ref, w_ref, H, W)
    _store_out_and_stats(acc, H, W, y_ref, st_ref)


def _head_kernel(prev_ref, ss_ref, p_ref, out_ref):
    act = jnp.maximum(
        prev_ref[...].astype(jnp.float32) * ss_ref[0:1, :] + ss_ref[1:2, :],
        0.0)
    out_ref[...] = jnp.dot(p_ref[...], act, preferred_element_type=jnp.float32)


def _bn_scale_shift(st, gamma, beta, cnt):
    N, _, C = st.shape
    return pl.pallas_call(
        partial(_ss_kernel, cnt=cnt),
        out_shape=jax.ShapeDtypeStruct((2, C), jnp.float32),
        compiler_params=pltpu.CompilerParams(vmem_limit_bytes=VMEM_LIMIT),
    )(st, gamma, beta)


def _conv_first(xpad_flat, wcol, H, W):
    N, Mp, Cin = xpad_flat.shape
    Cout = wcol.shape[-1]
    W2 = W + 2
    M = H * W2
    return pl.pallas_call(
        partial(_conv1_kernel, H=H, W=W),
        out_shape=(jax.ShapeDtypeStruct((N, M, Cout), jnp.float32),
                   jax.ShapeDtypeStruct((N, 2, Cout), jnp.float32)),
        grid=(N,),
        in_specs=[pl.BlockSpec((None, Mp, Cin), lambda n: (n, 0, 0)),
                  pl.BlockSpec((9 * Cin, Cout), lambda n: (0, 0))],
        out_specs=(pl.BlockSpec((None, M, Cout), lambda n: (n, 0, 0)),
                   pl.BlockSpec((None, 2, Cout), lambda n: (n, 0, 0))),
        scratch_shapes=[pltpu.VMEM((M, 9 * Cin), jnp.bfloat16)],
        compiler_params=pltpu.CompilerParams(
            dimension_semantics=("parallel",), vmem_limit_bytes=VMEM_LIMIT),
    )(xpad_flat, wcol)


def _fused_pool_conv(y_prev, ss, wcol, Hp, Wp):
    N = y_prev.shape[0]
    Cp = y_prev.shape[-1]
    Cout = wcol.shape[-1]
    W2p = Wp + 2
    Hc, Wc = (Hp - 3) // 2 + 1, (Wp - 3) // 2 + 1
    W2c = Wc + 2
    M = Hc * W2c
    prev4 = y_prev.reshape(N, Hp, W2p, Cp)
    return pl.pallas_call(
        partial(_pool_conv_kernel, Hp=Hp, Wp=Wp, Hc=Hc, Wc=Wc),
        out_shape=(jax.ShapeDtypeStruct((N, M, Cout), jnp.float32),
                   jax.ShapeDtypeStruct((N, 2, Cout), jnp.float32)),
        grid=(N,),
        in_specs=[pl.BlockSpec((None, Hp, W2p, Cp), lambda n: (n, 0, 0, 0)),
                  pl.BlockSpec((2, Cp), lambda n: (0, 0)),
                  pl.BlockSpec((9 * Cp, Cout), lambda n: (0, 0))],
        out_specs=(pl.BlockSpec((None, M, Cout), lambda n: (n, 0, 0)),
                   pl.BlockSpec((None, 2, Cout), lambda n: (n, 0, 0))),
        scratch_shapes=[pltpu.VMEM((Hp, W2p, Cp), jnp.float32),
                        pltpu.VMEM((Hp, Wc, Cp), jnp.float32),
                        pltpu.VMEM(((Hc + 3) * W2c, Cp), jnp.bfloat16),
                        pltpu.VMEM((M, 9 * Cp), jnp.bfloat16)],
        compiler_params=pltpu.CompilerParams(
            dimension_semantics=("parallel",), vmem_limit_bytes=VMEM_LIMIT),
    )(prev4, ss, wcol), Hc, Wc


def _fused_conv(y_prev, ss, wcol, H, W):
    N = y_prev.shape[0]
    Cp = y_prev.shape[-1]
    Cout = wcol.shape[-1]
    W2 = W + 2
    M = H * W2
    return pl.pallas_call(
        partial(_conv_kernel, H=H, W=W),
        out_shape=(jax.ShapeDtypeStruct((N, M, Cout), jnp.float32),
                   jax.ShapeDtypeStruct((N, 2, Cout), jnp.float32)),
        grid=(N,),
        in_specs=[pl.BlockSpec((None, M, Cp), lambda n: (n, 0, 0)),
                  pl.BlockSpec((2, Cp), lambda n: (0, 0)),
                  pl.BlockSpec((9 * Cp, Cout), lambda n: (0, 0))],
        out_specs=(pl.BlockSpec((None, M, Cout), lambda n: (n, 0, 0)),
                   pl.BlockSpec((None, 2, Cout), lambda n: (n, 0, 0))),
        scratch_shapes=[pltpu.VMEM(((H + 3) * W2, Cp), jnp.bfloat16),
                        pltpu.VMEM((M, 9 * Cp), jnp.bfloat16)],
        compiler_params=pltpu.CompilerParams(
            dimension_semantics=("parallel",), vmem_limit_bytes=VMEM_LIMIT),
    )(y_prev, ss, wcol)


def _head_pool(y_prev, ss, pmat):
    N, M, C = y_prev.shape
    P = pmat.shape[0]
    return pl.pallas_call(
        _head_kernel,
        out_shape=jax.ShapeDtypeStruct((N, P, C), jnp.float32),
        grid=(N,),
        in_specs=[pl.BlockSpec((None, M, C), lambda n: (n, 0, 0)),
                  pl.BlockSpec((2, C), lambda n: (0, 0)),
                  pl.BlockSpec((P, M), lambda n: (0, 0))],
        out_specs=pl.BlockSpec((None, P, C), lambda n: (n, 0, 0)),
        compiler_params=pltpu.CompilerParams(
            dimension_semantics=("parallel",), vmem_limit_bytes=VMEM_LIMIT),
    )(y_prev, ss, pmat)


def kernel(conv_w_0, conv_w_1, conv_w_2, conv_w_3, conv_w_4, conv_w_5,
           gamma_0, gamma_1, gamma_2, gamma_3, gamma_4, gamma_5,
           beta_0, beta_1, beta_2, beta_3, beta_4, beta_5,
           fc_w_perm, fc_b, pool_mat, x):
    conv_w = [conv_w_0, conv_w_1, conv_w_2, conv_w_3, conv_w_4, conv_w_5]
    gammas = [gamma_0, gamma_1, gamma_2, gamma_3, gamma_4, gamma_5]
    betas = [beta_0, beta_1, beta_2, beta_3, beta_4, beta_5]
    wcols = [w.reshape(w.shape[0] * w.shape[1], w.shape[2]) for w in conv_w]

    N, _, H, W = x.shape
    xh = jnp.transpose(x, (0, 2, 3, 1)).astype(jnp.float32)
    xp = jnp.pad(xh, ((0, 0), (1, 2), (1, 1), (0, 0))).astype(jnp.bfloat16)
    xp = xp.reshape(N, (H + 3) * (W + 2), x.shape[1])

    y, st = _conv_first(xp, wcols[0], H, W)
    h, w = H, W

    for i in (1, 2):
        ss = _bn_scale_shift(st, gammas[i - 1], betas[i - 1], float(N * h * w))
        (y, st), h, w = _fused_pool_conv(y, ss, wcols[i], h, w)

    for i in (3, 4, 5):
        ss = _bn_scale_shift(st, gammas[i - 1], betas[i - 1], float(N * h * w))
        y, st = _fused_conv(y, ss, wcols[i], h, w)

    ss = _bn_scale_shift(st, gammas[5], betas[5], float(N * h * w))
    pooled = _head_pool(y, ss, pool_mat)

    flat = pooled.reshape(N, -1)
    return flat @ fc_w_perm + fc_b[None, :]
